# baseline (speedup 1.0000x reference)
import functools

import jax
import jax.numpy as jnp
from jax import lax
from jax.experimental import pallas as pl
from jax.experimental.pallas import tpu as pltpu
from jax.experimental.pallas import tpu_sc as plsc

EPS = 1e-15
MAXN = 1.0 - 1e-5
NGRAPH = 64
NC, NS = 2, 16
NW = NC * NS
CHUNK = 128
BROWS = 2000


def _norm(x):
    return jnp.maximum(jnp.sqrt(jnp.sum(x * x, axis=-1, keepdims=True)), EPS)


def _artanh(x):
    x = jnp.clip(x, -1.0 + 1e-7, 1.0 - 1e-7)
    return 0.5 * jnp.log((1.0 + x) / (1.0 - x))


def _proj(x):
    n = _norm(x)
    return jnp.where(n > MAXN, x / n * MAXN, x)


def _expmap0(u):
    n = _norm(u)
    return jnp.tanh(n) * u / n


def _logmap0(p):
    n = _norm(p)
    return _artanh(n) * p / n


def _mobius_add(x, y):
    x2 = jnp.sum(x * x, axis=-1, keepdims=True)
    y2 = jnp.sum(y * y, axis=-1, keepdims=True)
    xy = jnp.sum(x * y, axis=-1, keepdims=True)
    num = (1.0 + 2.0 * xy + y2) * x + (1.0 - x2) * y
    den = 1.0 + 2.0 * xy + x2 * y2
    return num / jnp.maximum(den, EPS)


def _hyp_linear_t(x, wt, b):
    mx = jnp.dot(x, wt, preferred_element_type=jnp.float32)
    xn = _norm(x)
    mxn = _norm(mx)
    mv = jnp.tanh(mxn / xn * _artanh(xn)) * mx / mxn
    hb = _proj(_expmap0(b))
    return _proj(_mobius_add(_proj(mv), hb))


def _post_agg(p0, p1, d0, d1):
    deg = jnp.maximum(d0 + d1, 1.0)
    return _proj(_expmap0((p0 + p1) / deg))


def _lin_first_body(x_ref, wt_ref, b_ref, v_ref):
    h = _proj(_expmap0(x_ref[...]))
    h = _hyp_linear_t(h, wt_ref[...], b_ref[...])
    v_ref[...] = _logmap0(h)


def _lin_mid_body(p0_ref, p1_ref, d0_ref, d1_ref, wt_ref, b_ref, v_ref):
    h = _post_agg(p0_ref[...], p1_ref[...], d0_ref[...], d1_ref[...])
    t = _logmap0(h)
    t = jnp.where(t >= 0, t, 0.2 * t)
    h = _proj(_expmap0(t))
    h = _hyp_linear_t(h, wt_ref[...], b_ref[...])
    v_ref[...] = _logmap0(h)


def _pool_body(p0_ref, p1_ref, d0_ref, d1_ref, batch_ref, sums_ref, cnts_ref):
    i = pl.program_id(0)
    h = _post_agg(p0_ref[...], p1_ref[...], d0_ref[...], d1_ref[...])
    t = _logmap0(h)
    b = batch_ref[...]
    g = lax.broadcasted_iota(jnp.int32, (NGRAPH, t.shape[0]), 0)
    m = (g == b[:, 0][None, :]).astype(jnp.float32)

    @pl.when(i == 0)
    def _():
        sums_ref[...] = jnp.zeros_like(sums_ref)
        cnts_ref[...] = jnp.zeros_like(cnts_ref)

    sums_ref[...] += jnp.dot(m, t, preferred_element_type=jnp.float32)
    cnts_ref[...] += jnp.sum(m, axis=1, keepdims=True)


def _final_body(sums_ref, cnts_ref, wt_ref, b_ref, out_ref):
    pooled = sums_ref[...] / jnp.maximum(cnts_ref[...], 1.0)
    z = _expmap0(pooled)
    out_ref[...] = _hyp_linear_t(z, wt_ref[...], b_ref[...])


def _spread_zeros(zbuf, acc_sh, row0, n_per_tile):
    nfull = n_per_tile // CHUNK
    rem = n_per_tile - nfull * CHUNK
    for k in range(nfull):
        pltpu.sync_copy(zbuf, acc_sh.at[pl.ds(row0 + k * CHUNK, CHUNK)])
    if rem:
        pltpu.sync_copy(zbuf.at[pl.ds(0, rem)],
                        acc_sh.at[pl.ds(row0 + nfull * CHUNK, rem)])


@functools.lru_cache(maxsize=None)
def _make_deg_kernel(n_acc, c0, c1):
    n_per_tile = n_acc // NS
    cmax = max(c0, c1)
    mesh = plsc.VectorSubcoreMesh(core_axis_name="c", subcore_axis_name="s")

    @functools.partial(
        pl.kernel,
        out_type=jax.ShapeDtypeStruct((NC, n_acc, 128), jnp.float32),
        mesh=mesh,
        scratch_types=[
            pltpu.VMEM((cmax, CHUNK), jnp.int32),
            pltpu.VMEM((CHUNK, 128), jnp.float32),
            pltpu.VMEM_SHARED((n_acc, 128), jnp.float32),
            pltpu.SemaphoreType.DMA,
        ],
    )
    def deg_kernel(dst_hbm, zeros_hbm, ones_hbm, out_hbm,
                   idx_v, ones_v, acc_sh, ssem):
        cid = lax.axis_index("c")
        sid = lax.axis_index("s")
        row0 = sid * n_per_tile
        pltpu.sync_copy(zeros_hbm, ones_v)
        _spread_zeros(ones_v, acc_sh, row0, n_per_tile)
        pltpu.sync_copy(ones_hbm, ones_v)
        plsc.subcore_barrier()

        def run(base_row, nch):
            pltpu.sync_copy(dst_hbm.at[pl.ds(base_row, nch)],
                            idx_v.at[pl.ds(0, nch)])

            def group(g, carry):
                j0 = g * 8
                for s in range(8):
                    pltpu.async_copy(ones_v, acc_sh.at[idx_v.at[j0 + s]],
                                     ssem, add=True)
                for s in range(8):
                    pltpu.make_async_copy(ones_v, acc_sh.at[idx_v.at[j0 + s]],
                                          ssem).wait()
                return carry

            lax.fori_loop(0, nch // 8, group, 0)

        @pl.when(cid == 0)
        def _():
            run(sid * c0, c0)

        @pl.when(cid == 1)
        def _():
            run(NS * c0 + sid * c1, c1)

        plsc.subcore_barrier()
        pltpu.sync_copy(acc_sh.at[pl.ds(row0, n_per_tile)],
                        out_hbm.at[cid, pl.ds(row0, n_per_tile)])

    return deg_kernel


@functools.lru_cache(maxsize=None)
def _make_agg_kernel(n_acc, d, c0, c1):
    assert c0 == c1
    n_per_tile = n_acc // NS
    mesh = plsc.VectorSubcoreMesh(core_axis_name="c", subcore_axis_name="s")

    @functools.partial(
        pl.kernel,
        out_type=jax.ShapeDtypeStruct((NC, n_acc, d), jnp.float32),
        mesh=mesh,
        scratch_types=[
            pltpu.VMEM((CHUNK,), jnp.int32),
            pltpu.VMEM((CHUNK,), jnp.int32),
            pltpu.VMEM((CHUNK, d), jnp.float32),
            pltpu.VMEM_SHARED((n_acc, d), jnp.float32),
            pltpu.SemaphoreType.DMA,
        ],
    )
    def agg_kernel(v_hbm, src_hbm, dst_hbm, zeros_hbm, out_hbm,
                   src_c, dst_c, rows, acc_sh, gsem):
        cid = lax.axis_index("c")
        sid = lax.axis_index("s")
        row0 = sid * n_per_tile
        pltpu.sync_copy(zeros_hbm, rows)
        _spread_zeros(rows, acc_sh, row0, n_per_tile)
        plsc.subcore_barrier()
        base_row = (cid * NS + sid) * c0

        def body(j, carry):
            pltpu.sync_copy(src_hbm.at[base_row + j], src_c)
            pltpu.sync_copy(dst_hbm.at[base_row + j], dst_c)
            pltpu.async_copy(v_hbm.at[src_c], rows, gsem).wait()
            pltpu.sync_copy(rows, acc_sh.at[dst_c], add=True)
            return carry

        lax.fori_loop(0, c0, body, 0)
        plsc.subcore_barrier()
        pltpu.sync_copy(acc_sh.at[pl.ds(row0, n_per_tile)],
                        out_hbm.at[cid, pl.ds(row0, n_per_tile)])

    return agg_kernel


def _row_specs(n_blocks):
    del n_blocks
    full = pl.BlockSpec((BROWS, 128), lambda i: (i, 0))
    col = pl.BlockSpec((BROWS, 1), lambda i: (i, 0))
    w = pl.BlockSpec((128, 128), lambda i: (0, 0))
    b = pl.BlockSpec((1, 128), lambda i: (0, 0))
    return full, col, w, b


def kernel(x, edge_index, batch, W1, b1, W2, b2, W3, b3, W4, b4):
    n, d = x.shape
    e = edge_index.shape[1]
    src = edge_index[0].astype(jnp.int32)
    dst = edge_index[1].astype(jnp.int32)

    npair = -(-e // (NS * CHUNK))
    npair = max(32, ((npair + 31) // 32) * 32)
    c0 = npair // 2
    c1 = npair - c0
    e_pad = NS * npair * CHUNK
    n_acc = ((n + NS * 8) // (NS * 8)) * (NS * 8)
    if e_pad > e:
        src = jnp.concatenate([src, jnp.zeros((e_pad - e,), jnp.int32)])
        dst = jnp.concatenate([dst, jnp.full((e_pad - e,), n, jnp.int32)])
    src = src.reshape(NS * npair, CHUNK)
    dst = dst.reshape(NS * npair, CHUNK)

    zeros_d = jnp.zeros((CHUNK, d), jnp.float32)
    ones_d = jnp.ones((CHUNK, 128), jnp.float32)
    batch_c = batch.astype(jnp.int32).reshape(n, 1)

    n_blocks = n // BROWS
    full, col, wspec, bspec = _row_specs(n_blocks)

    deg_p = _make_deg_kernel(n_acc, npair // 2, npair - npair // 2)(dst, zeros_d, ones_d)
    d0 = deg_p[0, :n, 0:1]
    d1 = deg_p[1, :n, 0:1]

    agg = _make_agg_kernel(n_acc, d, c0, c1)

    v1 = pl.pallas_call(
        _lin_first_body,
        grid=(n_blocks,),
        in_specs=[full, wspec, bspec],
        out_specs=full,
        out_shape=jax.ShapeDtypeStruct((n, d), jnp.float32),
    )(x, W1.T, b1.reshape(1, d))

    p = agg(v1, src, dst, zeros_d)
    v2 = pl.pallas_call(
        _lin_mid_body,
        grid=(n_blocks,),
        in_specs=[full, full, col, col, wspec, bspec],
        out_specs=full,
        out_shape=jax.ShapeDtypeStruct((n, d), jnp.float32),
    )(p[0, :n], p[1, :n], d0, d1, W2.T, b2.reshape(1, d))

    p = agg(v2, src, dst, zeros_d)
    v3 = pl.pallas_call(
        _lin_mid_body,
        grid=(n_blocks,),
        in_specs=[full, full, col, col, wspec, bspec],
        out_specs=full,
        out_shape=jax.ShapeDtypeStruct((n, d), jnp.float32),
    )(p[0, :n], p[1, :n], d0, d1, W3.T, b3.reshape(1, d))

    p = agg(v3, src, dst, zeros_d)
    sums, cnts = pl.pallas_call(
        _pool_body,
        grid=(n_blocks,),
        in_specs=[full, full, col, col,
                  pl.BlockSpec((BROWS, 1), lambda i: (i, 0))],
        out_specs=[pl.BlockSpec((NGRAPH, 128), lambda i: (0, 0)),
                   pl.BlockSpec((NGRAPH, 1), lambda i: (0, 0))],
        out_shape=[jax.ShapeDtypeStruct((NGRAPH, 128), jnp.float32),
                   jax.ShapeDtypeStruct((NGRAPH, 1), jnp.float32)],
    )(p[0, :n], p[1, :n], d0, d1, batch_c)

    out = pl.pallas_call(
        _final_body,
        grid=(1,),
        in_specs=[pl.BlockSpec((NGRAPH, 128), lambda i: (0, 0)),
                  pl.BlockSpec((NGRAPH, 1), lambda i: (0, 0)),
                  wspec, bspec],
        out_specs=pl.BlockSpec((NGRAPH, 128), lambda i: (0, 0)),
        out_shape=jax.ShapeDtypeStruct((NGRAPH, 128), jnp.float32),
    )(sums, cnts, W4.T, b4.reshape(1, d))

    return out


# speedup vs baseline: 1.3792x; 1.3792x over previous
import functools

import jax
import jax.numpy as jnp
from jax import lax
from jax.experimental import pallas as pl
from jax.experimental.pallas import tpu as pltpu
from jax.experimental.pallas import tpu_sc as plsc

EPS = 1e-15
MAXN = 1.0 - 1e-5
NGRAPH = 64
NC, NS = 2, 16
NW = NC * NS
CHUNK = 128
BROWS = 2000


def _norm(x):
    return jnp.maximum(jnp.sqrt(jnp.sum(x * x, axis=-1, keepdims=True)), EPS)


def _artanh(x):
    x = jnp.clip(x, -1.0 + 1e-7, 1.0 - 1e-7)
    return 0.5 * jnp.log((1.0 + x) / (1.0 - x))


def _proj(x):
    n = _norm(x)
    return jnp.where(n > MAXN, x / n * MAXN, x)


def _expmap0(u):
    n = _norm(u)
    return jnp.tanh(n) * u / n


def _logmap0(p):
    n = _norm(p)
    return _artanh(n) * p / n


def _mobius_add(x, y):
    x2 = jnp.sum(x * x, axis=-1, keepdims=True)
    y2 = jnp.sum(y * y, axis=-1, keepdims=True)
    xy = jnp.sum(x * y, axis=-1, keepdims=True)
    num = (1.0 + 2.0 * xy + y2) * x + (1.0 - x2) * y
    den = 1.0 + 2.0 * xy + x2 * y2
    return num / jnp.maximum(den, EPS)


def _hyp_linear_t(x, wt, b):
    mx = jnp.dot(x, wt, preferred_element_type=jnp.float32)
    xn = _norm(x)
    mxn = _norm(mx)
    mv = jnp.tanh(mxn / xn * _artanh(xn)) * mx / mxn
    hb = _proj(_expmap0(b))
    return _proj(_mobius_add(_proj(mv), hb))


def _post_agg(p0, p1, d0, d1):
    deg = jnp.maximum(d0 + d1, 1.0)
    return _proj(_expmap0((p0 + p1) / deg))


def _lin_first_body(x_ref, wt_ref, b_ref, v_ref):
    h = _proj(_expmap0(x_ref[...]))
    h = _hyp_linear_t(h, wt_ref[...], b_ref[...])
    v_ref[...] = _logmap0(h)


def _lin_mid_body(p0_ref, p1_ref, d0_ref, d1_ref, wt_ref, b_ref, v_ref):
    h = _post_agg(p0_ref[...], p1_ref[...], d0_ref[...], d1_ref[...])
    t = _logmap0(h)
    t = jnp.where(t >= 0, t, 0.2 * t)
    h = _proj(_expmap0(t))
    h = _hyp_linear_t(h, wt_ref[...], b_ref[...])
    v_ref[...] = _logmap0(h)


def _pool_body(p0_ref, p1_ref, d0_ref, d1_ref, batch_ref, sums_ref, cnts_ref):
    i = pl.program_id(0)
    h = _post_agg(p0_ref[...], p1_ref[...], d0_ref[...], d1_ref[...])
    t = _logmap0(h)
    b = batch_ref[...]
    g = lax.broadcasted_iota(jnp.int32, (NGRAPH, t.shape[0]), 0)
    m = (g == b[:, 0][None, :]).astype(jnp.float32)

    @pl.when(i == 0)
    def _():
        sums_ref[...] = jnp.zeros_like(sums_ref)
        cnts_ref[...] = jnp.zeros_like(cnts_ref)

    sums_ref[...] += jnp.dot(m, t, preferred_element_type=jnp.float32)
    cnts_ref[...] += jnp.sum(m, axis=1, keepdims=True)


def _final_body(sums_ref, cnts_ref, wt_ref, b_ref, out_ref):
    pooled = sums_ref[...] / jnp.maximum(cnts_ref[...], 1.0)
    z = _expmap0(pooled)
    out_ref[...] = _hyp_linear_t(z, wt_ref[...], b_ref[...])


@functools.lru_cache(maxsize=None)
def _make_deg_kernel(e_pad, n_acc):
    n_per_tile = n_acc // NS
    e_per_tile = e_pad // NW
    n_chunks = e_per_tile // CHUNK
    mesh = plsc.VectorSubcoreMesh(core_axis_name="c", subcore_axis_name="s")

    @functools.partial(
        pl.kernel,
        out_type=jax.ShapeDtypeStruct((NC, n_acc, 128), jnp.float32),
        mesh=mesh,
        scratch_types=[
            pltpu.VMEM((CHUNK,), jnp.int32),
            pltpu.VMEM((CHUNK, 128), jnp.float32),
            pltpu.VMEM_SHARED((n_acc, 128), jnp.float32),
        ],
    )
    def deg_kernel(dst_hbm, zeros_hbm, ones_hbm, out_hbm, dst_v, ones_v, acc_sh):
        cid = lax.axis_index("c")
        sid = lax.axis_index("s")
        wid = sid * NC + cid
        row0 = sid * n_per_tile
        pltpu.sync_copy(zeros_hbm.at[pl.ds(row0, n_per_tile)],
                        acc_sh.at[pl.ds(row0, n_per_tile)])
        pltpu.sync_copy(ones_hbm, ones_v)
        plsc.subcore_barrier()
        base = wid * e_per_tile

        def chunk(j, carry):
            pltpu.sync_copy(dst_hbm.at[pl.ds(base + j * CHUNK, CHUNK)], dst_v)
            pltpu.sync_copy(ones_v, acc_sh.at[dst_v], add=True)
            return carry

        lax.fori_loop(0, n_chunks, chunk, 0)
        plsc.subcore_barrier()
        pltpu.sync_copy(acc_sh.at[pl.ds(row0, n_per_tile)],
                        out_hbm.at[cid, pl.ds(row0, n_per_tile)])

    return deg_kernel


@functools.lru_cache(maxsize=None)
def _make_agg_kernel(e_pad, n_acc, d):
    n_per_tile = n_acc // NS
    e_per_tile = e_pad // NW
    n_chunks = e_per_tile // CHUNK
    mesh = plsc.VectorSubcoreMesh(core_axis_name="c", subcore_axis_name="s")

    @functools.partial(
        pl.kernel,
        out_type=jax.ShapeDtypeStruct((NC, n_acc, d), jnp.float32),
        mesh=mesh,
        scratch_types=[
            pltpu.VMEM((CHUNK,), jnp.int32),
            pltpu.VMEM((CHUNK,), jnp.int32),
            pltpu.VMEM((CHUNK, d), jnp.float32),
            pltpu.VMEM_SHARED((n_acc, d), jnp.float32),
            pltpu.SemaphoreType.DMA,
        ],
    )
    def agg_kernel(v_hbm, src_hbm, dst_hbm, zeros_hbm, out_hbm,
                   src_v, dst_v, rows_v, acc_sh, sem):
        cid = lax.axis_index("c")
        sid = lax.axis_index("s")
        wid = sid * NC + cid
        row0 = sid * n_per_tile
        pltpu.sync_copy(zeros_hbm.at[pl.ds(row0, n_per_tile)],
                        acc_sh.at[pl.ds(row0, n_per_tile)])
        plsc.subcore_barrier()
        base = wid * e_per_tile

        def chunk(j, carry):
            off = base + j * CHUNK
            pltpu.sync_copy(src_hbm.at[pl.ds(off, CHUNK)], src_v)
            pltpu.sync_copy(dst_hbm.at[pl.ds(off, CHUNK)], dst_v)
            pltpu.async_copy(v_hbm.at[src_v], rows_v, sem).wait()
            pltpu.sync_copy(rows_v, acc_sh.at[dst_v], add=True)
            return carry

        lax.fori_loop(0, n_chunks, chunk, 0)
        plsc.subcore_barrier()
        pltpu.sync_copy(acc_sh.at[pl.ds(row0, n_per_tile)],
                        out_hbm.at[cid, pl.ds(row0, n_per_tile)])

    return agg_kernel


def _row_specs(n_blocks):
    del n_blocks
    full = pl.BlockSpec((BROWS, 128), lambda i: (i, 0))
    col = pl.BlockSpec((BROWS, 1), lambda i: (i, 0))
    w = pl.BlockSpec((128, 128), lambda i: (0, 0))
    b = pl.BlockSpec((1, 128), lambda i: (0, 0))
    return full, col, w, b


def kernel(x, edge_index, batch, W1, b1, W2, b2, W3, b3, W4, b4):
    n, d = x.shape
    e = edge_index.shape[1]
    src = edge_index[0].astype(jnp.int32)
    dst = edge_index[1].astype(jnp.int32)

    epw = NW * CHUNK
    e_pad = ((e + epw - 1) // epw) * epw
    n_acc = ((n + NS * 8) // (NS * 8)) * (NS * 8)
    if e_pad > e:
        src = jnp.concatenate([src, jnp.zeros((e_pad - e,), jnp.int32)])
        dst = jnp.concatenate([dst, jnp.full((e_pad - e,), n, jnp.int32)])

    zeros_d = jnp.zeros((n_acc, d), jnp.float32)
    ones_d = jnp.ones((CHUNK, 128), jnp.float32)
    batch_c = batch.astype(jnp.int32).reshape(n, 1)

    n_blocks = n // BROWS
    full, col, wspec, bspec = _row_specs(n_blocks)

    deg_p = _make_deg_kernel(e_pad, n_acc)(dst, zeros_d, ones_d)
    d0 = deg_p[0, :n, 0:1]
    d1 = deg_p[1, :n, 0:1]

    agg = _make_agg_kernel(e_pad, n_acc, d)

    v1 = pl.pallas_call(
        _lin_first_body,
        grid=(n_blocks,),
        in_specs=[full, wspec, bspec],
        out_specs=full,
        out_shape=jax.ShapeDtypeStruct((n, d), jnp.float32),
    )(x, W1.T, b1.reshape(1, d))

    p = agg(v1, src, dst, zeros_d)
    v2 = pl.pallas_call(
        _lin_mid_body,
        grid=(n_blocks,),
        in_specs=[full, full, col, col, wspec, bspec],
        out_specs=full,
        out_shape=jax.ShapeDtypeStruct((n, d), jnp.float32),
    )(p[0, :n], p[1, :n], d0, d1, W2.T, b2.reshape(1, d))

    p = agg(v2, src, dst, zeros_d)
    v3 = pl.pallas_call(
        _lin_mid_body,
        grid=(n_blocks,),
        in_specs=[full, full, col, col, wspec, bspec],
        out_specs=full,
        out_shape=jax.ShapeDtypeStruct((n, d), jnp.float32),
    )(p[0, :n], p[1, :n], d0, d1, W3.T, b3.reshape(1, d))

    p = agg(v3, src, dst, zeros_d)
    sums, cnts = pl.pallas_call(
        _pool_body,
        grid=(n_blocks,),
        in_specs=[full, full, col, col,
                  pl.BlockSpec((BROWS, 1), lambda i: (i, 0))],
        out_specs=[pl.BlockSpec((NGRAPH, 128), lambda i: (0, 0)),
                   pl.BlockSpec((NGRAPH, 1), lambda i: (0, 0))],
        out_shape=[jax.ShapeDtypeStruct((NGRAPH, 128), jnp.float32),
                   jax.ShapeDtypeStruct((NGRAPH, 1), jnp.float32)],
    )(p[0, :n], p[1, :n], d0, d1, batch_c)

    out = pl.pallas_call(
        _final_body,
        grid=(1,),
        in_specs=[pl.BlockSpec((NGRAPH, 128), lambda i: (0, 0)),
                  pl.BlockSpec((NGRAPH, 1), lambda i: (0, 0)),
                  wspec, bspec],
        out_specs=pl.BlockSpec((NGRAPH, 128), lambda i: (0, 0)),
        out_shape=jax.ShapeDtypeStruct((NGRAPH, 128), jnp.float32),
    )(sums, cnts, W4.T, b4.reshape(1, d))

    return out
